# Initial kernel scaffold; baseline (speedup 1.0000x reference)
import jax
import jax.numpy as jnp
from jax.experimental import pallas as pl


def kernel(e, edge_index, gamma, beta):
    raise NotImplementedError("write your pallas kernel here")


# speedup vs baseline: 6.2400x; 6.2400x over previous
import functools

import jax
import jax.numpy as jnp
from jax import lax
from jax.experimental import pallas as pl
from jax.experimental.pallas import tpu as pltpu
from jax.experimental.pallas import tpu_sc as plsc

N = 10000
E = 320000
D = 128
EPS = 1e-05

NC = 2
NS = 16
NTILES = NC * NS
EPT = E // NTILES
CH = 125
CW = 80
BLK = 3200
NBLK = E // BLK


def _rowsum_body(e_ref, s1_ref, s2_ref):
    x = e_ref[...]
    s1_ref[...] = jnp.sum(x, axis=1, keepdims=True)
    s2_ref[...] = jnp.sum(x * x, axis=1, keepdims=True)


def _rowsums(e):
    return pl.pallas_call(
        _rowsum_body,
        grid=(NBLK,),
        in_specs=[pl.BlockSpec((BLK, D), lambda i: (i, 0))],
        out_specs=[
            pl.BlockSpec((BLK, 1), lambda i: (i, 0)),
            pl.BlockSpec((BLK, 1), lambda i: (i, 0)),
        ],
        out_shape=[
            jax.ShapeDtypeStruct((E, 1), jnp.float32),
            jax.ShapeDtypeStruct((E, 1), jnp.float32),
        ],
    )(e)


def _sc_scatter(s1c, s2c, dstc):
    mesh = plsc.VectorSubcoreMesh(core_axis_name="c", subcore_axis_name="s")

    @functools.partial(
        pl.kernel,
        mesh=mesh,
        out_type=jax.ShapeDtypeStruct((NC * 3 * N,), jnp.float32),
        scratch_types=[
            pltpu.VMEM((CH, CW), jnp.int32),
            pltpu.VMEM((CH, CW), jnp.float32),
            pltpu.VMEM((CH, CW), jnp.float32),
            pltpu.VMEM((CW,), jnp.float32),
            pltpu.VMEM((N,), jnp.float32),
            pltpu.VMEM_SHARED((N,), jnp.float32),
            pltpu.VMEM_SHARED((N,), jnp.float32),
            pltpu.VMEM_SHARED((N,), jnp.float32),
        ],
    )
    def k(s1_hbm, s2_hbm, dst_hbm, out_hbm,
          dst_v, s1_v, s2_v, ones_v, z_v, a1, a2, a3):
        c = lax.axis_index("c")
        s = lax.axis_index("s")
        wid = c * NS + s
        pltpu.sync_copy(dst_hbm.at[wid], dst_v)
        pltpu.sync_copy(s1_hbm.at[wid], s1_v)
        pltpu.sync_copy(s2_hbm.at[wid], s2_v)

        def fill_ones(i, carry):
            ones_v[pl.ds(i * 16, 16)] = jnp.ones((16,), jnp.float32)
            return carry

        lax.fori_loop(0, CW // 16, fill_ones, 0)

        @pl.when(s == 0)
        def _():
            def zloop(i, carry):
                z_v[pl.ds(i * 16, 16)] = jnp.zeros((16,), jnp.float32)
                return carry

            lax.fori_loop(0, N // 16, zloop, 0)
            pltpu.sync_copy(z_v, a1)
            pltpu.sync_copy(z_v, a2)
            pltpu.sync_copy(z_v, a3)

        plsc.subcore_barrier()

        def chunk(kk, carry):
            idx = dst_v.at[kk]
            pltpu.sync_copy(s1_v.at[kk], a1.at[idx], add=True)
            pltpu.sync_copy(s2_v.at[kk], a2.at[idx], add=True)
            pltpu.sync_copy(ones_v, a3.at[idx], add=True)
            return carry

        lax.fori_loop(0, CH, chunk, 0)

        plsc.subcore_barrier()

        @pl.when(s == 0)
        def _():
            for r, acc in enumerate((a1, a2, a3)):
                pltpu.sync_copy(acc, z_v)
                pltpu.sync_copy(z_v, out_hbm.at[pl.ds((c * 3 + r) * N, N)])

    return k(s1c, s2c, dstc)


def _sc_gather(partials, dstg):
    mesh = plsc.VectorSubcoreMesh(core_axis_name="c", subcore_axis_name="s")

    @functools.partial(
        pl.kernel,
        mesh=mesh,
        out_type=jax.ShapeDtypeStruct((3 * E,), jnp.float32),
        compiler_params=pltpu.CompilerParams(needs_layout_passes=False),
        scratch_types=[
            pltpu.VMEM((N,), jnp.float32),
            pltpu.VMEM((N,), jnp.float32),
            pltpu.VMEM((N,), jnp.float32),
            pltpu.VMEM((N,), jnp.float32),
            pltpu.VMEM((EPT,), jnp.int32),
            pltpu.VMEM((3 * EPT,), jnp.float32),
        ],
    )
    def k(p_hbm, dst_hbm, out_hbm, pa1, pa2, pa3, pb, dst_v, out_v):
        c = lax.axis_index("c")
        s = lax.axis_index("s")
        wid = c * NS + s
        pltpu.sync_copy(dst_hbm.at[pl.ds(wid * EPT, EPT)], dst_v)
        for r, pr in enumerate((pa1, pa2, pa3)):
            pltpu.sync_copy(p_hbm.at[pl.ds(r * N, N)], pr)
            pltpu.sync_copy(p_hbm.at[pl.ds((3 + r) * N, N)], pb)

            def comb(i, carry, pr=pr):
                sl = pl.ds(i * 16, 16)
                pr[sl] = pr[sl] + pb[sl]
                return carry

            lax.fori_loop(0, N // 16, comb, 0)

        lane = lax.iota(jnp.int32, 16)

        def chunk(j, carry):
            idx = dst_v[pl.ds(j * 16, 16)]
            g1 = plsc.load_gather(pa1, [idx])
            g2 = plsc.load_gather(pa2, [idx])
            g3 = plsc.load_gather(pa3, [idx])
            p3 = (j * 16 + lane) * 3
            plsc.store_scatter(out_v, [p3], g1)
            plsc.store_scatter(out_v, [p3 + 1], g2)
            plsc.store_scatter(out_v, [p3 + 2], g3)
            return carry

        lax.fori_loop(0, EPT // 16, chunk, 0)
        pltpu.sync_copy(out_v, out_hbm.at[pl.ds(wid * 3 * EPT, 3 * EPT)])

    return k(partials, dstg)


def _final_body(e_ref, mi_ref, g_ref, b_ref, o_ref):
    x = e_ref[...]
    s1 = mi_ref[:, 0:1]
    s2 = mi_ref[:, 1:2]
    dg = mi_ref[:, 2:3]
    count = dg * D
    mean = s1 / count
    ss = jnp.maximum(s2 - count * mean * mean, 0.0)
    var = ss / jnp.maximum(count - 1.0, 1.0)
    inv = 1.0 / (jnp.sqrt(var) + EPS)
    o_ref[...] = g_ref[...] * ((x - mean) * inv) + b_ref[...]


def _final(e, mi, gamma, beta):
    return pl.pallas_call(
        _final_body,
        grid=(NBLK,),
        in_specs=[
            pl.BlockSpec((BLK, D), lambda i: (i, 0)),
            pl.BlockSpec((BLK, 3), lambda i: (i, 0)),
            pl.BlockSpec((1, D), lambda i: (0, 0)),
            pl.BlockSpec((1, D), lambda i: (0, 0)),
        ],
        out_specs=pl.BlockSpec((BLK, D), lambda i: (i, 0)),
        out_shape=jax.ShapeDtypeStruct((E, D), jnp.float32),
    )(e, mi, gamma, beta)


def kernel(e, edge_index, gamma, beta):
    dst = edge_index[1].astype(jnp.int32)
    s1, s2 = _rowsums(e)
    s1c = s1.reshape(NTILES, CH, CW)
    s2c = s2.reshape(NTILES, CH, CW)
    dstc = dst.reshape(NTILES, CH, CW)
    partials = _sc_scatter(s1c, s2c, dstc)
    mi = _sc_gather(partials, dst).reshape(E, 3)
    return _final(e, mi, gamma.reshape(1, D), beta.reshape(1, D))


# speedup vs baseline: 7.2695x; 1.1650x over previous
import functools

import jax
import jax.numpy as jnp
from jax import lax
from jax.experimental import pallas as pl
from jax.experimental.pallas import tpu as pltpu
from jax.experimental.pallas import tpu_sc as plsc

N = 10000
E = 320000
D = 128
EPS = 1e-05

NC = 2
NS = 16
EPS_T = E // NS
CW = 128
CH = (EPS_T + CW - 1) // CW
PADW = CH * CW - EPS_T
GRP = 5
NP = 10240
NSL = NP // NS
BLK = 3200
NBLK = E // BLK


def _rowsum_body(e_ref, s1_ref, s2_ref):
    x = e_ref[...]
    s1_ref[...] = jnp.sum(x, axis=1, keepdims=True)
    s2_ref[...] = jnp.sum(x * x, axis=1, keepdims=True)


def _rowsums(e):
    return pl.pallas_call(
        _rowsum_body,
        grid=(NBLK,),
        in_specs=[pl.BlockSpec((BLK, D), lambda i: (i, 0))],
        out_specs=[
            pl.BlockSpec((BLK, 1), lambda i: (i, 0)),
            pl.BlockSpec((BLK, 1), lambda i: (i, 0)),
        ],
        out_shape=[
            jax.ShapeDtypeStruct((E, 1), jnp.float32),
            jax.ShapeDtypeStruct((E, 1), jnp.float32),
        ],
    )(e)


def _sc_stats(s1g, s2g, dstg):
    mesh = plsc.VectorSubcoreMesh(core_axis_name="c", subcore_axis_name="s")

    @functools.partial(
        pl.kernel,
        mesh=mesh,
        out_type=jax.ShapeDtypeStruct((2 * E,), jnp.float32),
        compiler_params=pltpu.CompilerParams(needs_layout_passes=False),
        scratch_types=[
            pltpu.VMEM((CH, CW), jnp.int32),
            pltpu.VMEM((CH, CW), jnp.float32),
            pltpu.VMEM((CH, CW), jnp.float32),
            pltpu.VMEM((CW,), jnp.float32),
            pltpu.VMEM((NSL,), jnp.float32),
            pltpu.VMEM((NSL,), jnp.float32),
            pltpu.VMEM((NSL,), jnp.float32),
            pltpu.VMEM((NSL,), jnp.float32),
            pltpu.VMEM((NSL,), jnp.float32),
            pltpu.VMEM((NP,), jnp.float32),
            pltpu.VMEM((NP,), jnp.float32),
            pltpu.VMEM((EPS_T,), jnp.float32),
            pltpu.VMEM_SHARED((NP,), jnp.float32),
            pltpu.VMEM_SHARED((NP,), jnp.float32),
            pltpu.VMEM_SHARED((NP,), jnp.float32),
            pltpu.VMEM_SHARED((NP,), jnp.float32),
            pltpu.VMEM_SHARED((NP,), jnp.float32),
            pltpu.SemaphoreType.DMA,
        ],
    )
    def k(s1_hbm, s2_hbm, dst_hbm, out_hbm,
          dst_v, s1_v, s2_v, ones_v, u1, u2, u3, msl, ivsl,
          m_v, iv_v, out_v, a1, a2, a3, m_sh, iv_sh, sem):
        c = lax.axis_index("c")
        s = lax.axis_index("s")
        pltpu.sync_copy(dst_hbm.at[s], dst_v)
        pltpu.sync_copy(s1_hbm.at[s], s1_v)
        pltpu.sync_copy(s2_hbm.at[s], s2_v)

        def fill_ones(i, carry):
            ones_v[pl.ds(i * 16, 16)] = jnp.ones((16,), jnp.float32)
            return carry

        lax.fori_loop(0, CW // 16, fill_ones, 0)

        def zloop(i, carry):
            u1[pl.ds(i * 16, 16)] = jnp.zeros((16,), jnp.float32)
            return carry

        lax.fori_loop(0, NSL // 16, zloop, 0)
        nbase = s * NSL
        pltpu.sync_copy(u1, a1.at[pl.ds(nbase, NSL)])
        pltpu.sync_copy(u1, a2.at[pl.ds(nbase, NSL)])
        pltpu.sync_copy(u1, a3.at[pl.ds(nbase, NSL)])
        plsc.subcore_barrier()

        def fire(kk):
            idx = dst_v.at[kk]
            return [
                pltpu.async_copy(s1_v.at[kk], a1.at[idx], sem, add=True),
                pltpu.async_copy(s2_v.at[kk], a2.at[idx], sem, add=True),
                pltpu.async_copy(ones_v, a3.at[idx], sem, add=True),
            ]

        def sgroup(i, carry):
            cps = []
            for j in range(GRP):
                cps.extend(fire(i * GRP + j))
            for cp in cps:
                cp.wait()
            return carry

        lax.fori_loop(0, CH // GRP, sgroup, 0)
        cps = []
        for kk in range((CH // GRP) * GRP, CH):
            cps.extend(fire(kk))
        for cp in cps:
            cp.wait()
        plsc.subcore_barrier()

        pltpu.sync_copy(a1.at[pl.ds(nbase, NSL)], u1)
        pltpu.sync_copy(a2.at[pl.ds(nbase, NSL)], u2)
        pltpu.sync_copy(a3.at[pl.ds(nbase, NSL)], u3)
        magic = jnp.full((16,), 0x5F3759DF, dtype=jnp.int32)

        def stat(i, carry):
            sl = pl.ds(i * 16, 16)
            count = u3[sl] * float(D)
            mean = u1[sl] / jnp.maximum(count, 1.0)
            ss = jnp.maximum(u2[sl] - count * mean * mean, 0.0)
            var = ss / jnp.maximum(count - 1.0, 1.0)
            var = jnp.maximum(var, 1e-30)
            vi = plsc.bitcast(var, jnp.int32)
            r = plsc.bitcast(magic - lax.shift_right_logical(vi, 1),
                             jnp.float32)
            half = -0.5 * var
            r = r * (1.5 + half * r * r)
            r = r * (1.5 + half * r * r)
            r = r * (1.5 + half * r * r)
            std = var * r
            msl[sl] = mean
            ivsl[sl] = 1.0 / (std + EPS)
            return carry

        lax.fori_loop(0, NSL // 16, stat, 0)
        pltpu.sync_copy(msl, m_sh.at[pl.ds(nbase, NSL)])
        pltpu.sync_copy(ivsl, iv_sh.at[pl.ds(nbase, NSL)])
        plsc.subcore_barrier()

        pltpu.sync_copy(m_sh, m_v)
        pltpu.sync_copy(iv_sh, iv_v)
        lane = lax.iota(jnp.int32, 16)

        def chunk(t, carry):
            p = c * (EPS_T // NC) + t * 16
            row = p // CW
            col = p % CW
            idx = dst_v[row, pl.ds(col, 16)]
            gm = plsc.load_gather(m_v, [idx])
            giv = plsc.load_gather(iv_v, [idx])
            p2 = t * 32 + lane * 2
            plsc.store_scatter(out_v, [p2], gm)
            plsc.store_scatter(out_v, [p2 + 1], giv)
            return carry

        lax.fori_loop(0, (EPS_T // NC) // 16, chunk, 0)
        obase = s * (2 * EPS_T) + c * EPS_T
        pltpu.sync_copy(out_v, out_hbm.at[pl.ds(obase, EPS_T)])

    return k(s1g, s2g, dstg)


def _final_body(e_ref, mi_ref, g_ref, b_ref, o_ref):
    x = e_ref[...]
    mean = mi_ref[:, 0:1]
    inv = mi_ref[:, 1:2]
    o_ref[...] = g_ref[...] * ((x - mean) * inv) + b_ref[...]


def _final(e, mi, gamma, beta):
    return pl.pallas_call(
        _final_body,
        grid=(NBLK,),
        in_specs=[
            pl.BlockSpec((BLK, D), lambda i: (i, 0)),
            pl.BlockSpec((BLK, 2), lambda i: (i, 0)),
            pl.BlockSpec((1, D), lambda i: (0, 0)),
            pl.BlockSpec((1, D), lambda i: (0, 0)),
        ],
        out_specs=pl.BlockSpec((BLK, D), lambda i: (i, 0)),
        out_shape=jax.ShapeDtypeStruct((E, D), jnp.float32),
    )(e, mi, gamma, beta)


def kernel(e, edge_index, gamma, beta):
    dst = edge_index[1].astype(jnp.int32)
    s1, s2 = _rowsums(e)
    pad0 = ((0, 0), (0, PADW))
    s1g = jnp.pad(s1.reshape(NS, EPS_T), pad0).reshape(NS, CH, CW)
    s2g = jnp.pad(s2.reshape(NS, EPS_T), pad0).reshape(NS, CH, CW)
    dstg = jnp.pad(dst.reshape(NS, EPS_T), pad0,
                   constant_values=N).reshape(NS, CH, CW)
    mi = _sc_stats(s1g, s2g, dstg).reshape(E, 2)
    return _final(e, mi, gamma.reshape(1, D), beta.reshape(1, D))


# speedup vs baseline: 14.2867x; 1.9653x over previous
import functools

import jax
import jax.numpy as jnp
from jax import lax
from jax.experimental import pallas as pl
from jax.experimental.pallas import tpu as pltpu
from jax.experimental.pallas import tpu_sc as plsc

N = 10000
E = 320000
D = 128
EPS = 1e-05

NC = 2
NS = 16
CW = 128
ROWS = E // CW
CH = 157
PR = CH * NS
P = PR * CW
EPT = P // NS
GPT = EPT // NC
GRP = 5
NP = 10240
NSL = NP // NS
BLK = 3200
BR = BLK // CW
NBLK = E // BLK


def _rowsum_body(e_ref, s1_ref, s2_ref):
    x4 = e_ref[...].reshape(1, BR, CW, D)
    s1_ref[...] = jnp.sum(x4, axis=3)
    s2_ref[...] = jnp.sum(x4 * x4, axis=3)


def _rowsums(e):
    return pl.pallas_call(
        _rowsum_body,
        grid=(NBLK,),
        in_specs=[pl.BlockSpec((BLK, D), lambda i: (i, 0))],
        out_specs=[
            pl.BlockSpec((1, BR, CW), lambda i: (i, 0, 0)),
            pl.BlockSpec((1, BR, CW), lambda i: (i, 0, 0)),
        ],
        out_shape=[
            jax.ShapeDtypeStruct((NBLK, BR, CW), jnp.float32),
            jax.ShapeDtypeStruct((NBLK, BR, CW), jnp.float32),
        ],
    )(e)


def _sc_stats(s1g, s2g, dstg):
    mesh = plsc.VectorSubcoreMesh(core_axis_name="c", subcore_axis_name="s")

    @functools.partial(
        pl.kernel,
        mesh=mesh,
        out_type=[
            jax.ShapeDtypeStruct((P,), jnp.float32),
            jax.ShapeDtypeStruct((P,), jnp.float32),
        ],
        compiler_params=pltpu.CompilerParams(needs_layout_passes=False),
        scratch_types=[
            pltpu.VMEM((CH, CW), jnp.int32),
            pltpu.VMEM((CH, CW), jnp.float32),
            pltpu.VMEM((CH, CW), jnp.float32),
            pltpu.VMEM((CW,), jnp.float32),
            pltpu.VMEM((NSL,), jnp.float32),
            pltpu.VMEM((NSL,), jnp.float32),
            pltpu.VMEM((NSL,), jnp.float32),
            pltpu.VMEM((NSL,), jnp.float32),
            pltpu.VMEM((NSL,), jnp.float32),
            pltpu.VMEM((NP,), jnp.float32),
            pltpu.VMEM((NP,), jnp.float32),
            pltpu.VMEM((GPT,), jnp.float32),
            pltpu.VMEM((GPT,), jnp.float32),
            pltpu.VMEM_SHARED((NP,), jnp.float32),
            pltpu.VMEM_SHARED((NP,), jnp.float32),
            pltpu.VMEM_SHARED((NP,), jnp.float32),
            pltpu.VMEM_SHARED((NP,), jnp.float32),
            pltpu.VMEM_SHARED((NP,), jnp.float32),
            pltpu.SemaphoreType.DMA,
        ],
    )
    def k(s1_hbm, s2_hbm, dst_hbm, m_hbm, iv_hbm,
          dst_v, s1_v, s2_v, ones_v, u1, u2, u3, msl, ivsl,
          m_v, iv_v, om_v, oiv_v, a1, a2, a3, m_sh, iv_sh, sem):
        c = lax.axis_index("c")
        s = lax.axis_index("s")
        pltpu.sync_copy(dst_hbm.at[s], dst_v)
        pltpu.sync_copy(s1_hbm.at[s], s1_v)
        pltpu.sync_copy(s2_hbm.at[s], s2_v)

        def fill_ones(i, carry):
            ones_v[pl.ds(i * 16, 16)] = jnp.ones((16,), jnp.float32)
            return carry

        lax.fori_loop(0, CW // 16, fill_ones, 0)

        def zloop(i, carry):
            u1[pl.ds(i * 16, 16)] = jnp.zeros((16,), jnp.float32)
            return carry

        lax.fori_loop(0, NSL // 16, zloop, 0)
        nbase = s * NSL
        pltpu.sync_copy(u1, a1.at[pl.ds(nbase, NSL)])
        pltpu.sync_copy(u1, a2.at[pl.ds(nbase, NSL)])
        pltpu.sync_copy(u1, a3.at[pl.ds(nbase, NSL)])
        plsc.subcore_barrier()

        def fire(kk):
            idx = dst_v.at[kk]
            return [
                pltpu.async_copy(s1_v.at[kk], a1.at[idx], sem, add=True),
                pltpu.async_copy(s2_v.at[kk], a2.at[idx], sem, add=True),
                pltpu.async_copy(ones_v, a3.at[idx], sem, add=True),
            ]

        def sgroup(i, carry):
            cps = []
            for j in range(GRP):
                cps.extend(fire(i * GRP + j))
            for cp in cps:
                cp.wait()
            return carry

        lax.fori_loop(0, CH // GRP, sgroup, 0)
        cps = []
        for kk in range((CH // GRP) * GRP, CH):
            cps.extend(fire(kk))
        for cp in cps:
            cp.wait()
        plsc.subcore_barrier()

        pltpu.sync_copy(a1.at[pl.ds(nbase, NSL)], u1)
        pltpu.sync_copy(a2.at[pl.ds(nbase, NSL)], u2)
        pltpu.sync_copy(a3.at[pl.ds(nbase, NSL)], u3)
        magic = jnp.full((16,), 0x5F3759DF, dtype=jnp.int32)

        def stat(i, carry):
            sl = pl.ds(i * 16, 16)
            count = u3[sl] * float(D)
            mean = u1[sl] / jnp.maximum(count, 1.0)
            ss = jnp.maximum(u2[sl] - count * mean * mean, 0.0)
            var = ss / jnp.maximum(count - 1.0, 1.0)
            var = jnp.maximum(var, 1e-30)
            vi = plsc.bitcast(var, jnp.int32)
            r = plsc.bitcast(magic - lax.shift_right_logical(vi, 1),
                             jnp.float32)
            half = -0.5 * var
            r = r * (1.5 + half * r * r)
            r = r * (1.5 + half * r * r)
            r = r * (1.5 + half * r * r)
            std = var * r
            msl[sl] = mean
            ivsl[sl] = 1.0 / (std + EPS)
            return carry

        lax.fori_loop(0, NSL // 16, stat, 0)
        pltpu.sync_copy(msl, m_sh.at[pl.ds(nbase, NSL)])
        pltpu.sync_copy(ivsl, iv_sh.at[pl.ds(nbase, NSL)])
        plsc.subcore_barrier()

        pltpu.sync_copy(m_sh, m_v)
        pltpu.sync_copy(iv_sh, iv_v)

        def chunk(t, carry):
            p = c * GPT + t * 16
            row = p // CW
            col = p % CW
            idx = dst_v[row, pl.ds(col, 16)]
            om_v[pl.ds(t * 16, 16)] = plsc.load_gather(m_v, [idx])
            oiv_v[pl.ds(t * 16, 16)] = plsc.load_gather(iv_v, [idx])
            return carry

        lax.fori_loop(0, GPT // 16, chunk, 0)
        obase = s * EPT + c * GPT
        pltpu.sync_copy(om_v, m_hbm.at[pl.ds(obase, GPT)])
        pltpu.sync_copy(oiv_v, iv_hbm.at[pl.ds(obase, GPT)])

    return k(s1g, s2g, dstg)


def _final_body(e_ref, m_ref, iv_ref, g_ref, b_ref, o_ref):
    x4 = e_ref[...].reshape(1, BR, CW, D)
    mean = lax.broadcast_in_dim(m_ref[...], (1, BR, CW, D), (0, 1, 2))
    inv = lax.broadcast_in_dim(iv_ref[...], (1, BR, CW, D), (0, 1, 2))
    g4 = lax.broadcast_in_dim(g_ref[...], (1, BR, CW, D), (0, 3))
    b4 = lax.broadcast_in_dim(b_ref[...], (1, BR, CW, D), (0, 3))
    o_ref[...] = (g4 * ((x4 - mean) * inv) + b4).reshape(BLK, D)


def _final(e, m2, iv2, gamma, beta):
    return pl.pallas_call(
        _final_body,
        grid=(NBLK,),
        in_specs=[
            pl.BlockSpec((BLK, D), lambda i: (i, 0)),
            pl.BlockSpec((1, BR, CW), lambda i: (i, 0, 0)),
            pl.BlockSpec((1, BR, CW), lambda i: (i, 0, 0)),
            pl.BlockSpec((1, D), lambda i: (0, 0)),
            pl.BlockSpec((1, D), lambda i: (0, 0)),
        ],
        out_specs=pl.BlockSpec((BLK, D), lambda i: (i, 0)),
        out_shape=jax.ShapeDtypeStruct((E, D), jnp.float32),
    )(e, m2, iv2, gamma, beta)


def kernel(e, edge_index, gamma, beta):
    dst = edge_index[1].astype(jnp.int32)
    s1d, s2d = _rowsums(e)
    padr = ((0, PR - ROWS), (0, 0))
    s1g = jnp.pad(s1d.reshape(ROWS, CW), padr).reshape(NS, CH, CW)
    s2g = jnp.pad(s2d.reshape(ROWS, CW), padr).reshape(NS, CH, CW)
    dstg = jnp.pad(dst.reshape(ROWS, CW), padr,
                   constant_values=N).reshape(NS, CH, CW)
    m1, iv1 = _sc_stats(s1g, s2g, dstg)
    m2 = m1[:E].reshape(NBLK, BR, CW)
    iv2 = iv1[:E].reshape(NBLK, BR, CW)
    return _final(e, m2, iv2, gamma.reshape(1, D), beta.reshape(1, D))
